# baseline (speedup 1.0000x reference)
import functools

import jax
import jax.numpy as jnp
from jax import lax
from jax.experimental import pallas as pl
from jax.experimental.pallas import tpu as pltpu
from jax.experimental.pallas import tpu_sc as plsc

VOCAB = 76
DIM = 768
B, L = 64, 1024

NC, NS = 2, 16
NW = NC * NS
ROWS = B * L
ROWS_PER_W = ROWS // NW
CHUNK = 16
NCHUNK = ROWS_PER_W // CHUNK
NBUF = 4
NGROUP = NCHUNK // NBUF
VPAD = 80
LANES = 16


def _sc_lookup(ids_hbm, table_hbm, out_hbm, idx_v, table_v, bufs, so):
    wid = lax.axis_index("s") * NC + lax.axis_index("c")
    base = wid * ROWS_PER_W * DIM
    pltpu.sync_copy(table_hbm, table_v)
    pltpu.sync_copy(ids_hbm.at[wid], idx_v)

    lane_iota = lax.iota(jnp.int32, LANES)

    def writeout(c, b):
        return pltpu.make_async_copy(
            bufs[b], out_hbm.at[pl.ds(base + c * (CHUNK * DIM), CHUNK * DIM)], so[b]
        )

    def group_body(j, carry):
        for b in range(NBUF):
            c = j * NBUF + b

            @pl.when(c >= NBUF)
            def _():
                writeout(c - NBUF, b).wait()

            ids16 = idx_v[pl.ds(c * CHUNK, CHUNK)]
            rowbase = ids16 * DIM

            @plsc.parallel_loop(0, CHUNK, 1, unroll=2)
            def row_body(r, b=b):
                rb = jnp.sum(jnp.where(lane_iota == r, rowbase, 0))
                vals = [table_v[pl.ds(rb + k * LANES, LANES)] for k in range(DIM // LANES)]
                for k in range(DIM // LANES):
                    bufs[b][pl.ds(r * DIM + k * LANES, LANES)] = vals[k]

            writeout(c, b).start()

        return carry

    lax.fori_loop(0, NGROUP, group_body, 0)
    for b in range(NBUF):
        writeout(NCHUNK - NBUF + b, b).wait()


@functools.partial(jax.jit, static_argnames=())
def kernel(phoneme_ids, table):
    ids = phoneme_ids.astype(jnp.int32).reshape(NW, ROWS_PER_W)
    table_pad = (
        jnp.zeros((VPAD, DIM), jnp.float32).at[:VOCAB].set(table).reshape(VPAD * DIM)
    )
    mesh = plsc.VectorSubcoreMesh(
        core_axis_name="c", subcore_axis_name="s", num_cores=NC, num_subcores=NS
    )
    out = pl.kernel(
        _sc_lookup,
        out_type=jax.ShapeDtypeStruct((ROWS * DIM,), jnp.float32),
        mesh=mesh,
        compiler_params=pltpu.CompilerParams(needs_layout_passes=False),
        scratch_types=[
            pltpu.VMEM((ROWS_PER_W,), jnp.int32),
            pltpu.VMEM((VPAD * DIM,), jnp.float32),
            [pltpu.VMEM((CHUNK * DIM,), jnp.float32) for _ in range(NBUF)],
            [pltpu.SemaphoreType.DMA for _ in range(NBUF)],
        ],
    )(ids, table_pad)
    embeds = out.reshape(B, L, DIM)
    mask = jnp.ones((B, L), dtype=jnp.float32)
    return (embeds, mask)


# speedup vs baseline: 1.2535x; 1.2535x over previous
import functools

import jax
import jax.numpy as jnp
from jax import lax
from jax.experimental import pallas as pl
from jax.experimental.pallas import tpu as pltpu
from jax.experimental.pallas import tpu_sc as plsc

VOCAB = 76
DIM = 768
B, L = 64, 1024

NC, NS = 2, 16
NW = NC * NS
ROWS = B * L
ROWS_PER_W = ROWS // NW
CHUNK = 16
NCHUNK = ROWS_PER_W // CHUNK
NBUF = 4
NGROUP = NCHUNK // NBUF
VPAD = 128
STAGE = VPAD // NS
LANES = 16


def _sc_lookup(ids_hbm, table_hbm, out_hbm, idx_v, table_sp, bufs, sg, so):
    sid = lax.axis_index("s")
    wid = sid * NC + lax.axis_index("c")
    base = wid * ROWS_PER_W * DIM
    nstage = STAGE * DIM
    pltpu.sync_copy(table_hbm.at[pl.ds(sid * nstage, nstage)], bufs[0].at[pl.ds(0, nstage)])
    pltpu.sync_copy(bufs[0].at[pl.ds(0, nstage)], table_sp.at[pl.ds(sid * nstage, nstage)])
    pltpu.sync_copy(ids_hbm.at[wid], idx_v)
    plsc.subcore_barrier()

    lane_iota = lax.iota(jnp.int32, LANES)

    def writeout(c, b):
        return pltpu.make_async_copy(
            bufs[b], out_hbm.at[pl.ds(base + c * (CHUNK * DIM), CHUNK * DIM)], so[b]
        )

    def row_fetch(rb, r, b):
        return pltpu.make_async_copy(
            table_sp.at[pl.ds(rb, DIM)], bufs[b].at[pl.ds(r * DIM, DIM)], sg[b]
        )

    def group_body(j, carry):
        for b in range(NBUF):
            c = j * NBUF + b

            @pl.when(c >= NBUF)
            def _():
                writeout(c - NBUF, b).wait()

            ids16 = idx_v[pl.ds(c * CHUNK, CHUNK)]
            rowbase = ids16 * DIM
            fetches = []
            for r in range(CHUNK):
                rb = pl.multiple_of(jnp.sum(jnp.where(lane_iota == r, rowbase, 0)), 8)
                h = row_fetch(rb, r, b)
                h.start()
                fetches.append(h)
            for h in fetches:
                h.wait()
            writeout(c, b).start()

        return carry

    lax.fori_loop(0, NGROUP, group_body, 0)
    for b in range(NBUF):
        writeout(NCHUNK - NBUF + b, b).wait()


@functools.partial(jax.jit, static_argnames=())
def kernel(phoneme_ids, table):
    ids = phoneme_ids.astype(jnp.int32).reshape(NW, ROWS_PER_W)
    table_pad = (
        jnp.zeros((VPAD, DIM), jnp.float32).at[:VOCAB].set(table).reshape(VPAD * DIM)
    )
    mesh = plsc.VectorSubcoreMesh(
        core_axis_name="c", subcore_axis_name="s", num_cores=NC, num_subcores=NS
    )
    out = pl.kernel(
        _sc_lookup,
        out_type=jax.ShapeDtypeStruct((ROWS * DIM,), jnp.float32),
        mesh=mesh,
        compiler_params=pltpu.CompilerParams(needs_layout_passes=False),
        scratch_types=[
            pltpu.VMEM((ROWS_PER_W,), jnp.int32),
            pltpu.VMEM_SHARED((VPAD * DIM,), jnp.float32),
            [pltpu.VMEM((CHUNK * DIM,), jnp.float32) for _ in range(NBUF)],
            [pltpu.SemaphoreType.DMA for _ in range(NBUF)],
            [pltpu.SemaphoreType.DMA for _ in range(NBUF)],
        ],
    )(ids, table_pad)
    embeds = out.reshape(B, L, DIM)
    mask = jnp.ones((B, L), dtype=jnp.float32)
    return (embeds, mask)


# speedup vs baseline: 2.2552x; 1.7992x over previous
import functools

import jax
import jax.numpy as jnp
from jax import lax
from jax.experimental import pallas as pl
from jax.experimental.pallas import tpu as pltpu
from jax.experimental.pallas import tpu_sc as plsc

VOCAB = 76
DIM = 768
B, L = 64, 1024

NC, NS = 2, 16
NW = NC * NS
ROWS = B * L
ROWS_PER_W = ROWS // NW
CHUNK = 32
NCHUNK = ROWS_PER_W // CHUNK
NBUF = 4
NGROUP = NCHUNK // NBUF


def _sc_gather(ids_hbm, table_hbm, out_hbm, idx_v, bufs, sg, so):
    wid = lax.axis_index("s") * NC + lax.axis_index("c")
    base = wid * ROWS_PER_W
    pltpu.sync_copy(ids_hbm.at[wid], idx_v)

    def gather(c, b):
        return pltpu.make_async_copy(table_hbm.at[idx_v.at[c]], bufs[b], sg[b])

    def writeout(c, b):
        return pltpu.make_async_copy(
            bufs[b], out_hbm.at[pl.ds(base + c * CHUNK, CHUNK)], so[b]
        )

    for b in range(NBUF - 1):
        gather(b, b).start()

    def group_body(j, carry):
        for b in range(NBUF):
            c = j * NBUF + b
            gather(c, b).wait()
            writeout(c, b).start()
            bp = (b - 1) % NBUF

            @pl.when(c >= 1)
            def _():
                writeout(c - 1, bp).wait()

            @pl.when(c + NBUF - 1 < NCHUNK)
            def _():
                gather(c + NBUF - 1, bp).start()

        return carry

    lax.fori_loop(0, NGROUP, group_body, 0)
    writeout(NCHUNK - 1, NBUF - 1).wait()


@functools.partial(jax.jit, static_argnames=())
def kernel(phoneme_ids, table):
    ids = phoneme_ids.astype(jnp.int32).reshape(NW, NCHUNK, CHUNK)
    ids = ids + (jnp.arange(NW, dtype=jnp.int32) * VOCAB)[:, None, None]
    table_rep = jnp.broadcast_to(table, (NW,) + table.shape).reshape(NW * VOCAB, DIM)
    mesh = plsc.VectorSubcoreMesh(
        core_axis_name="c", subcore_axis_name="s", num_cores=NC, num_subcores=NS
    )
    out = pl.kernel(
        _sc_gather,
        out_type=jax.ShapeDtypeStruct((ROWS, DIM), jnp.float32),
        mesh=mesh,
        scratch_types=[
            pltpu.VMEM((NCHUNK, CHUNK), jnp.int32),
            [pltpu.VMEM((CHUNK, DIM), jnp.float32) for _ in range(NBUF)],
            [pltpu.SemaphoreType.DMA for _ in range(NBUF)],
            [pltpu.SemaphoreType.DMA for _ in range(NBUF)],
        ],
    )(ids, table_rep)
    embeds = out.reshape(B, L, DIM)
    mask = jnp.ones((B, L), dtype=jnp.float32)
    return (embeds, mask)
